# baseline (speedup 1.0000x reference)
import functools

import jax
import jax.numpy as jnp
from jax import lax
from jax.experimental import pallas as pl
from jax.experimental.pallas import tpu as pltpu
from jax.experimental.pallas import tpu_sc as plsc

EMBED = 32
NROWS = 1000000
BATCH = 16384
NC = 2
NS = 16
NW = NC * NS

LANE_TILES = -(-NROWS // 128)
TILES_PER_W = LANE_TILES // NW
TILES_REM = LANE_TILES % NW
NCHUNKS = -(-(TILES_PER_W + 1) * 128 // 1024)
CHUNK_COLS = 1024
LIST_CAP = 1536
CLIST_CAP = 256
OUT_ROWS = BATCH + 16
SENTINEL = -(2 ** 20)


def _gather_body(uid_hbm, iid_hbm, ut_hbm, it_hbm, ue_hbm, ie_hbm,
                 ids_v, lid_v, lpos_v, clid_v, clpos_v, chunk_v, rows_v,
                 semr):
    wid = lax.axis_index("s") * NC + lax.axis_index("c")
    tiles_lo = wid * TILES_PER_W + jnp.minimum(wid, TILES_REM)
    ntiles = TILES_PER_W + (wid < TILES_REM).astype(jnp.int32)
    cols_lo = tiles_lo * 128
    cols_hi = (tiles_lo + ntiles) * 128
    lanes = lax.iota(jnp.int32, 16)
    d_lo = lanes
    d_hi = lanes + 16

    def extract(vec, l):
        return jnp.sum(jnp.where(lanes == l, vec, 0))

    def scan_ids(ids_hbm, loff):
        cur = jnp.int32(loff)
        for b in range(BATCH // 1024):
            pltpu.sync_copy(ids_hbm.at[pl.ds(b * 8, 8)], ids_v)

            def sbody(t, cur):
                v = ids_v[t // 8, pl.ds((t % 8) * 16, 16)]
                p = b * 1024 + t * 16 + lanes
                m = (v >= cols_lo) & (v < cols_hi)
                plsc.store_compressed(lid_v.at[pl.ds(cur, 16)], v, mask=m)
                plsc.store_compressed(lpos_v.at[pl.ds(cur, 16)], p, mask=m)
                return cur + jnp.max(plsc.all_reduce_population_count(m))

            cur = lax.fori_loop(0, 64, sbody, cur)
        lid_v[pl.ds(cur, 16)] = jnp.full((16,), SENTINEL, jnp.int32)
        return cur

    def phase(ids_hbm, tab_hbm, out_hbm, loff, gg):
        n = scan_ids(ids_hbm, loff)
        nvec = (n - loff + 15) // 16

        def chunk_body(k, gg):
            clo = cols_lo + k * CHUNK_COLS
            chi = jnp.minimum(clo + CHUNK_COLS, cols_hi)
            cstart = pl.multiple_of(jnp.minimum(clo, cols_hi - CHUNK_COLS),
                                    128)
            pltpu.sync_copy(tab_hbm.at[:, pl.ds(cstart, CHUNK_COLS)], chunk_v)

            def mbody(t, ccur):
                v = lid_v[pl.ds(loff + t * 16, 16)]
                p = lpos_v[pl.ds(loff + t * 16, 16)]
                m = (v >= clo) & (v < chi)
                plsc.store_compressed(clid_v.at[pl.ds(ccur, 16)], v - cstart,
                                      mask=m)
                plsc.store_compressed(clpos_v.at[pl.ds(ccur, 16)], p, mask=m)
                return ccur + jnp.max(plsc.all_reduce_population_count(m))

            ccur = lax.fori_loop(0, nvec, mbody, jnp.int32(0))
            clid_v[pl.ds(ccur, 16)] = jnp.zeros((16,), jnp.int32)
            clpos_v[pl.ds(ccur, 16)] = jnp.full((16,), BATCH, jnp.int32)
            ngroups = (ccur + 15) // 16

            def gbody(g, gg):
                parity = (gg % 2) * 16
                @pl.when(gg >= 2)
                def _():
                    pltpu.make_async_copy(
                        out_hbm.at[pl.ds(0, 16)],
                        rows_v.at[pl.ds(parity, 16)], semr).wait()

                colv = clid_v[pl.ds(g * 16, 16)]
                posv = clpos_v[pl.ds(g * 16, 16)]
                for l in range(16):
                    col = extract(colv, l)
                    pos = extract(posv, l)
                    cvec = jnp.full((16,), col, jnp.int32)
                    s = parity + l
                    rows_v[s, pl.ds(0, 16)] = plsc.load_gather(
                        chunk_v, [d_lo, cvec])
                    rows_v[s, pl.ds(16, 16)] = plsc.load_gather(
                        chunk_v, [d_hi, cvec])
                    pltpu.async_copy(rows_v.at[s], out_hbm.at[pos], semr)
                return gg + 1

            return lax.fori_loop(0, ngroups, gbody, gg)

        gg = lax.fori_loop(0, NCHUNKS, chunk_body, gg)
        return gg

    gg = phase(uid_hbm, ut_hbm, ue_hbm, 0, jnp.int32(0))
    gg = phase(iid_hbm, it_hbm, ie_hbm, LIST_CAP, gg)
    @pl.when(gg >= 1)
    def _():
        pltpu.make_async_copy(ue_hbm.at[pl.ds(0, 16)],
                              rows_v.at[pl.ds(((gg - 1) % 2) * 16, 16)],
                              semr).wait()

    @pl.when(gg >= 2)
    def _():
        pltpu.make_async_copy(ue_hbm.at[pl.ds(0, 16)],
                              rows_v.at[pl.ds((gg % 2) * 16, 16)],
                              semr).wait()


@functools.cache
def _gather():
    return pl.kernel(
        _gather_body,
        out_type=(
            jax.ShapeDtypeStruct((OUT_ROWS, EMBED), jnp.float32),
            jax.ShapeDtypeStruct((OUT_ROWS, EMBED), jnp.float32),
        ),
        mesh=plsc.VectorSubcoreMesh(core_axis_name="c", subcore_axis_name="s",
                                    num_cores=NC, num_subcores=NS),
        scratch_types=[
            pltpu.VMEM((8, 128), jnp.int32),
            pltpu.VMEM((2 * LIST_CAP,), jnp.int32),
            pltpu.VMEM((2 * LIST_CAP,), jnp.int32),
            pltpu.VMEM((CLIST_CAP,), jnp.int32),
            pltpu.VMEM((CLIST_CAP,), jnp.int32),
            pltpu.VMEM((EMBED, CHUNK_COLS), jnp.float32),
            pltpu.VMEM((32, EMBED), jnp.float32),
            pltpu.SemaphoreType.DMA,
        ],
        compiler_params=pltpu.CompilerParams(needs_layout_passes=False),
    )


BLK = 1024


def _mlp_body(xu_ref, xi_ref, w1u_ref, w1i_ref, b1_ref, w2_ref, b2_ref,
              w3_ref, b3_ref, out_ref):
    x1 = jnp.dot(xu_ref[...], w1u_ref[...], preferred_element_type=jnp.float32)
    x2 = jnp.dot(xi_ref[...], w1i_ref[...], preferred_element_type=jnp.float32)
    h = jnp.maximum(x1 + x2 + b1_ref[...], 0.0)
    h = jnp.maximum(
        jnp.dot(h, w2_ref[...], preferred_element_type=jnp.float32) + b2_ref[...],
        0.0)
    out_ref[...] = (
        jnp.dot(h, w3_ref[...], preferred_element_type=jnp.float32) + b3_ref[...])


def _mlp(xu, xi, w1u, w1i, b1, w2, b2, w3, b3):
    grid = (BATCH // BLK,)
    full = lambda shape: pl.BlockSpec(shape, lambda i: (0,) * len(shape))
    return pl.pallas_call(
        _mlp_body,
        grid=grid,
        in_specs=[
            pl.BlockSpec((BLK, EMBED), lambda i: (i, 0)),
            pl.BlockSpec((BLK, EMBED), lambda i: (i, 0)),
            full((EMBED, 256)),
            full((EMBED, 256)),
            full((1, 256)),
            full((256, 64)),
            full((1, 64)),
            full((64, 1)),
            full((1, 1)),
        ],
        out_specs=pl.BlockSpec((BLK, 1), lambda i: (i, 0)),
        out_shape=jax.ShapeDtypeStruct((BATCH, 1), jnp.float32),
    )(xu, xi, w1u, w1i, b1, w2, b2, w3, b3)


def kernel(user_id, item_id, user_table, item_table, W1, b1, W2, b2, W3, b3):
    uid = user_id.astype(jnp.int32).reshape(128, 128)
    iid = item_id.astype(jnp.int32).reshape(128, 128)
    ue, ie = _gather()(uid, iid, user_table.T, item_table.T)
    return _mlp(ue, ie, W1[:EMBED], W1[EMBED:], b1.reshape(1, 256),
                W2, b2.reshape(1, 64), W3, b3.reshape(1, 1))


# speedup vs baseline: 1.4943x; 1.4943x over previous
import functools

import jax
import jax.numpy as jnp
from jax import lax
from jax.experimental import pallas as pl
from jax.experimental.pallas import tpu as pltpu
from jax.experimental.pallas import tpu_sc as plsc

EMBED = 32
NROWS = 1000000
BATCH = 16384
NC = 2
NS = 16
NW = NC * NS

LANE_TILES = -(-NROWS // 128)
TILES_PER_W = LANE_TILES // NW
TILES_REM = LANE_TILES % NW
CHUNK_COLS = 2048
NCHUNKS = -(-(TILES_PER_W + 1) * 128 // CHUNK_COLS)
LIST_CAP = 1536
CLIST_CAP = 256
OUT_ROWS = BATCH + 16
SENTINEL = -(2 ** 20)


def _gather_body(uid_hbm, iid_hbm, ut_hbm, it_hbm, ue_hbm, ie_hbm,
                 ids_v, lid_v, lpos_v, clid_v, clpos_v, chunk_v, rows_v,
                 posbuf_v, semr):
    wid = lax.axis_index("s") * NC + lax.axis_index("c")
    tiles_lo = wid * TILES_PER_W + jnp.minimum(wid, TILES_REM)
    ntiles = TILES_PER_W + (wid < TILES_REM).astype(jnp.int32)
    cols_lo = tiles_lo * 128
    cols_hi = (tiles_lo + ntiles) * 128
    lanes = lax.iota(jnp.int32, 16)
    d_lo = lanes
    d_hi = lanes + 16

    def extract(vec, l):
        return jnp.sum(jnp.where(lanes == l, vec, 0))

    def scan_ids(ids_hbm, loff):
        cur = jnp.int32(loff)
        for b in range(BATCH // 1024):
            pltpu.sync_copy(ids_hbm.at[pl.ds(b * 8, 8)], ids_v)

            def sbody(t, cur):
                v = ids_v[t // 8, pl.ds((t % 8) * 16, 16)]
                p = b * 1024 + t * 16 + lanes
                m = (v >= cols_lo) & (v < cols_hi)
                plsc.store_compressed(lid_v.at[pl.ds(cur, 16)], v, mask=m)
                plsc.store_compressed(lpos_v.at[pl.ds(cur, 16)], p, mask=m)
                return cur + jnp.max(plsc.all_reduce_population_count(m))

            cur = lax.fori_loop(0, 64, sbody, cur)
        lid_v[pl.ds(cur, 16)] = jnp.full((16,), SENTINEL, jnp.int32)
        return cur

    def phase(ids_hbm, tab_hbm, out_hbm, loff, gg):
        n = scan_ids(ids_hbm, loff)
        nvec = (n - loff + 15) // 16

        def chunk_body(k, gg):
            clo = cols_lo + k * CHUNK_COLS
            chi = jnp.minimum(clo + CHUNK_COLS, cols_hi)
            cstart = pl.multiple_of(jnp.minimum(clo, cols_hi - CHUNK_COLS),
                                    128)
            pltpu.sync_copy(tab_hbm.at[:, pl.ds(cstart, CHUNK_COLS)], chunk_v)

            def mbody(t, ccur):
                v = lid_v[pl.ds(loff + t * 16, 16)]
                p = lpos_v[pl.ds(loff + t * 16, 16)]
                m = (v >= clo) & (v < chi)
                plsc.store_compressed(clid_v.at[pl.ds(ccur, 16)], v - cstart,
                                      mask=m)
                plsc.store_compressed(clpos_v.at[pl.ds(ccur, 16)], p, mask=m)
                return ccur + jnp.max(plsc.all_reduce_population_count(m))

            ccur = lax.fori_loop(0, nvec, mbody, jnp.int32(0))
            clid_v[pl.ds(ccur, 16)] = jnp.zeros((16,), jnp.int32)
            clpos_v[pl.ds(ccur, 16)] = jnp.full((16,), BATCH, jnp.int32)
            ngroups = (ccur + 15) // 16

            def gbody(g, gg):
                parity = gg % 2
                p16 = parity * 16
                @pl.when(gg >= 2)
                def _():
                    pltpu.make_async_copy(
                        out_hbm.at[pl.ds(0, 16)],
                        rows_v.at[pl.ds(p16, 16)], semr).wait()

                colv = clid_v[pl.ds(g * 16, 16)]
                posv = clpos_v[pl.ds(g * 16, 16)]
                for l in range(16):
                    col = extract(colv, l)
                    pos = extract(posv, l)
                    cvec = jnp.full((16,), col, jnp.int32)
                    s = p16 + l
                    rows_v[s, pl.ds(0, 16)] = plsc.load_gather(
                        chunk_v, [d_lo, cvec])
                    rows_v[s, pl.ds(16, 16)] = plsc.load_gather(
                        chunk_v, [d_hi, cvec])
                    pltpu.async_copy(rows_v.at[s], out_hbm.at[pos], semr)
                return gg + 1

            return lax.fori_loop(0, ngroups, gbody, gg)

        gg = lax.fori_loop(0, NCHUNKS, chunk_body, gg)
        return gg

    gg = phase(uid_hbm, ut_hbm, ue_hbm, 0, jnp.int32(0))
    gg = phase(iid_hbm, it_hbm, ie_hbm, LIST_CAP, gg)
    @pl.when(gg >= 1)
    def _():
        pltpu.make_async_copy(ue_hbm.at[pl.ds(0, 16)],
                              rows_v.at[pl.ds(((gg - 1) % 2) * 16, 16)],
                              semr).wait()

    @pl.when(gg >= 2)
    def _():
        pltpu.make_async_copy(ue_hbm.at[pl.ds(0, 16)],
                              rows_v.at[pl.ds((gg % 2) * 16, 16)],
                              semr).wait()


@functools.cache
def _gather():
    return pl.kernel(
        _gather_body,
        out_type=(
            jax.ShapeDtypeStruct((OUT_ROWS, EMBED), jnp.float32),
            jax.ShapeDtypeStruct((OUT_ROWS, EMBED), jnp.float32),
        ),
        mesh=plsc.VectorSubcoreMesh(core_axis_name="c", subcore_axis_name="s",
                                    num_cores=NC, num_subcores=NS),
        scratch_types=[
            pltpu.VMEM((8, 128), jnp.int32),
            pltpu.VMEM((2 * LIST_CAP,), jnp.int32),
            pltpu.VMEM((2 * LIST_CAP,), jnp.int32),
            pltpu.VMEM((CLIST_CAP,), jnp.int32),
            pltpu.VMEM((CLIST_CAP,), jnp.int32),
            pltpu.VMEM((EMBED, CHUNK_COLS), jnp.float32),
            pltpu.VMEM((32, EMBED), jnp.float32),
            pltpu.VMEM((2, 16), jnp.int32),
            pltpu.SemaphoreType.DMA,
        ],
        compiler_params=pltpu.CompilerParams(needs_layout_passes=False),
    )


BLK = 1024


def _mlp_body(xu_ref, xi_ref, w1u_ref, w1i_ref, b1_ref, w2_ref, b2_ref,
              w3_ref, b3_ref, out_ref):
    x1 = jnp.dot(xu_ref[...], w1u_ref[...], preferred_element_type=jnp.float32)
    x2 = jnp.dot(xi_ref[...], w1i_ref[...], preferred_element_type=jnp.float32)
    h = jnp.maximum(x1 + x2 + b1_ref[...], 0.0)
    h = jnp.maximum(
        jnp.dot(h, w2_ref[...], preferred_element_type=jnp.float32) + b2_ref[...],
        0.0)
    out_ref[...] = (
        jnp.dot(h, w3_ref[...], preferred_element_type=jnp.float32) + b3_ref[...])


def _mlp(xu, xi, w1u, w1i, b1, w2, b2, w3, b3):
    grid = (BATCH // BLK,)
    full = lambda shape: pl.BlockSpec(shape, lambda i: (0,) * len(shape))
    return pl.pallas_call(
        _mlp_body,
        grid=grid,
        in_specs=[
            pl.BlockSpec((BLK, EMBED), lambda i: (i, 0)),
            pl.BlockSpec((BLK, EMBED), lambda i: (i, 0)),
            full((EMBED, 256)),
            full((EMBED, 256)),
            full((1, 256)),
            full((256, 64)),
            full((1, 64)),
            full((64, 1)),
            full((1, 1)),
        ],
        out_specs=pl.BlockSpec((BLK, 1), lambda i: (i, 0)),
        out_shape=jax.ShapeDtypeStruct((BATCH, 1), jnp.float32),
    )(xu, xi, w1u, w1i, b1, w2, b2, w3, b3)


def kernel(user_id, item_id, user_table, item_table, W1, b1, W2, b2, W3, b3):
    uid = user_id.astype(jnp.int32).reshape(128, 128)
    iid = item_id.astype(jnp.int32).reshape(128, 128)
    ue, ie = _gather()(uid, iid, user_table.T, item_table.T)
    return _mlp(ue, ie, W1[:EMBED], W1[EMBED:], b1.reshape(1, 256),
                W2, b2.reshape(1, 64), W3, b3.reshape(1, 1))


# speedup vs baseline: 1.6734x; 1.1199x over previous
import functools

import jax
import jax.numpy as jnp
from jax import lax
from jax.experimental import pallas as pl
from jax.experimental.pallas import tpu as pltpu
from jax.experimental.pallas import tpu_sc as plsc

EMBED = 32
NROWS = 1000000
BATCH = 16384
NC = 2
NS = 16
NW = NC * NS

LANE_TILES = -(-NROWS // 128)
TILES_PER_W = LANE_TILES // NW
TILES_REM = LANE_TILES % NW
CHUNK_COLS = 1536
NCHUNKS = -(-(TILES_PER_W + 1) * 128 // CHUNK_COLS)
LIST_CAP = 1536
CLIST_CAP = 256
OUT_ROWS = BATCH + 16
SENTINEL = -(2 ** 20)


def _gather_body(uid_hbm, iid_hbm, ut_hbm, it_hbm, ue_hbm, ie_hbm,
                 ids_v, lid_v, lpos_v, clid_v, clpos_v, uchunk_v, ichunk_v,
                 rows_v, semu, semi, semr):
    wid = lax.axis_index("s") * NC + lax.axis_index("c")
    tiles_lo = wid * TILES_PER_W + jnp.minimum(wid, TILES_REM)
    ntiles = TILES_PER_W + (wid < TILES_REM).astype(jnp.int32)
    cols_lo = tiles_lo * 128
    cols_hi = (tiles_lo + ntiles) * 128
    lanes = lax.iota(jnp.int32, 16)
    d_lo = lanes
    d_hi = lanes + 16

    def extract(vec, l):
        return jnp.sum(jnp.where(lanes == l, vec, 0))

    def scan_ids(ids_hbm, loff):
        cur = jnp.int32(loff)
        for b in range(BATCH // 1024):
            pltpu.sync_copy(ids_hbm.at[pl.ds(b * 8, 8)], ids_v)

            def sbody(t, cur):
                v = ids_v[t // 8, pl.ds((t % 8) * 16, 16)]
                p = b * 1024 + t * 16 + lanes
                m = (v >= cols_lo) & (v < cols_hi)
                plsc.store_compressed(lid_v.at[pl.ds(cur, 16)], v, mask=m)
                plsc.store_compressed(lpos_v.at[pl.ds(cur, 16)], p, mask=m)
                return cur + jnp.max(plsc.all_reduce_population_count(m))

            cur = lax.fori_loop(0, 64, sbody, cur)
        lid_v[pl.ds(cur, 16)] = jnp.full((16,), SENTINEL, jnp.int32)
        return cur

    def start_chunk(tab_hbm, buf, sem, k):
        clo = cols_lo + k * CHUNK_COLS
        cstart = pl.multiple_of(jnp.minimum(clo, cols_hi - CHUNK_COLS), 128)
        pltpu.async_copy(tab_hbm.at[:, pl.ds(cstart, CHUNK_COLS)], buf, sem)

    def wait_chunk(tab_hbm, buf, sem):
        pltpu.make_async_copy(tab_hbm.at[:, pl.ds(0, CHUNK_COLS)], buf,
                              sem).wait()

    def process(chunk_v, loff, nvec, out_hbm, k, gg):
        clo = cols_lo + k * CHUNK_COLS
        chi = jnp.minimum(clo + CHUNK_COLS, cols_hi)
        cstart = pl.multiple_of(jnp.minimum(clo, cols_hi - CHUNK_COLS), 128)
        if True:
            def mbody(t, ccur):
                v = lid_v[pl.ds(loff + t * 16, 16)]
                p = lpos_v[pl.ds(loff + t * 16, 16)]
                m = (v >= clo) & (v < chi)
                plsc.store_compressed(clid_v.at[pl.ds(ccur, 16)], v - cstart,
                                      mask=m)
                plsc.store_compressed(clpos_v.at[pl.ds(ccur, 16)], p, mask=m)
                return ccur + jnp.max(plsc.all_reduce_population_count(m))

            ccur = lax.fori_loop(0, nvec, mbody, jnp.int32(0))
            clid_v[pl.ds(ccur, 16)] = jnp.zeros((16,), jnp.int32)
            clpos_v[pl.ds(ccur, 16)] = jnp.full((16,), BATCH, jnp.int32)
            ngroups = (ccur + 15) // 16

            def gbody(g, gg):
                parity = gg % 2
                p16 = parity * 16
                @pl.when(gg >= 2)
                def _():
                    pltpu.make_async_copy(
                        out_hbm.at[pl.ds(0, 16)],
                        rows_v.at[pl.ds(p16, 16)], semr).wait()

                colv = clid_v[pl.ds(g * 16, 16)]
                posv = clpos_v[pl.ds(g * 16, 16)]
                for l in range(16):
                    col = extract(colv, l)
                    pos = extract(posv, l)
                    cvec = jnp.full((16,), col, jnp.int32)
                    s = p16 + l
                    rows_v[s, pl.ds(0, 16)] = plsc.load_gather(
                        chunk_v, [d_lo, cvec])
                    rows_v[s, pl.ds(16, 16)] = plsc.load_gather(
                        chunk_v, [d_hi, cvec])
                    pltpu.async_copy(rows_v.at[s], out_hbm.at[pos], semr)
                return gg + 1

            return lax.fori_loop(0, ngroups, gbody, gg)

    nu = scan_ids(uid_hbm, 0)
    ni = scan_ids(iid_hbm, LIST_CAP)
    nvec_u = (nu + 15) // 16
    nvec_i = (ni - LIST_CAP + 15) // 16

    start_chunk(ut_hbm, uchunk_v, semu, jnp.int32(0))

    def chunk_body(k, gg):
        start_chunk(it_hbm, ichunk_v, semi, k)
        wait_chunk(ut_hbm, uchunk_v, semu)
        gg = process(uchunk_v, 0, nvec_u, ue_hbm, k, gg)

        @pl.when(k + 1 < NCHUNKS)
        def _():
            start_chunk(ut_hbm, uchunk_v, semu, k + 1)

        wait_chunk(it_hbm, ichunk_v, semi)
        gg = process(ichunk_v, LIST_CAP, nvec_i, ie_hbm, k, gg)
        return gg

    gg = lax.fori_loop(0, NCHUNKS, chunk_body, jnp.int32(0))
    @pl.when(gg >= 1)
    def _():
        pltpu.make_async_copy(ue_hbm.at[pl.ds(0, 16)],
                              rows_v.at[pl.ds(((gg - 1) % 2) * 16, 16)],
                              semr).wait()

    @pl.when(gg >= 2)
    def _():
        pltpu.make_async_copy(ue_hbm.at[pl.ds(0, 16)],
                              rows_v.at[pl.ds((gg % 2) * 16, 16)],
                              semr).wait()


@functools.cache
def _gather():
    return pl.kernel(
        _gather_body,
        out_type=(
            jax.ShapeDtypeStruct((OUT_ROWS, EMBED), jnp.float32),
            jax.ShapeDtypeStruct((OUT_ROWS, EMBED), jnp.float32),
        ),
        mesh=plsc.VectorSubcoreMesh(core_axis_name="c", subcore_axis_name="s",
                                    num_cores=NC, num_subcores=NS),
        scratch_types=[
            pltpu.VMEM((8, 128), jnp.int32),
            pltpu.VMEM((2 * LIST_CAP,), jnp.int32),
            pltpu.VMEM((2 * LIST_CAP,), jnp.int32),
            pltpu.VMEM((CLIST_CAP,), jnp.int32),
            pltpu.VMEM((CLIST_CAP,), jnp.int32),
            pltpu.VMEM((EMBED, CHUNK_COLS), jnp.float32),
            pltpu.VMEM((EMBED, CHUNK_COLS), jnp.float32),
            pltpu.VMEM((32, EMBED), jnp.float32),
            pltpu.SemaphoreType.DMA,
            pltpu.SemaphoreType.DMA,
            pltpu.SemaphoreType.DMA,
        ],
        compiler_params=pltpu.CompilerParams(needs_layout_passes=False),
    )


BLK = 1024


def _mlp_body(xu_ref, xi_ref, w1u_ref, w1i_ref, b1_ref, w2_ref, b2_ref,
              w3_ref, b3_ref, out_ref):
    x1 = jnp.dot(xu_ref[...], w1u_ref[...], preferred_element_type=jnp.float32)
    x2 = jnp.dot(xi_ref[...], w1i_ref[...], preferred_element_type=jnp.float32)
    h = jnp.maximum(x1 + x2 + b1_ref[...], 0.0)
    h = jnp.maximum(
        jnp.dot(h, w2_ref[...], preferred_element_type=jnp.float32) + b2_ref[...],
        0.0)
    out_ref[...] = (
        jnp.dot(h, w3_ref[...], preferred_element_type=jnp.float32) + b3_ref[...])


def _mlp(xu, xi, w1u, w1i, b1, w2, b2, w3, b3):
    grid = (BATCH // BLK,)
    full = lambda shape: pl.BlockSpec(shape, lambda i: (0,) * len(shape))
    return pl.pallas_call(
        _mlp_body,
        grid=grid,
        in_specs=[
            pl.BlockSpec((BLK, EMBED), lambda i: (i, 0)),
            pl.BlockSpec((BLK, EMBED), lambda i: (i, 0)),
            full((EMBED, 256)),
            full((EMBED, 256)),
            full((1, 256)),
            full((256, 64)),
            full((1, 64)),
            full((64, 1)),
            full((1, 1)),
        ],
        out_specs=pl.BlockSpec((BLK, 1), lambda i: (i, 0)),
        out_shape=jax.ShapeDtypeStruct((BATCH, 1), jnp.float32),
    )(xu, xi, w1u, w1i, b1, w2, b2, w3, b3)


def kernel(user_id, item_id, user_table, item_table, W1, b1, W2, b2, W3, b3):
    uid = user_id.astype(jnp.int32).reshape(128, 128)
    iid = item_id.astype(jnp.int32).reshape(128, 128)
    ue, ie = _gather()(uid, iid, user_table.T, item_table.T)
    return _mlp(ue, ie, W1[:EMBED], W1[EMBED:], b1.reshape(1, 256),
                W2, b2.reshape(1, 64), W3, b3.reshape(1, 1))
